# baseline (speedup 1.0000x reference)
import dataclasses
import functools

import jax
import jax.numpy as jnp
from jax import lax
from jax.experimental import pallas as pl
from jax.experimental.pallas import tpu as pltpu
from jax.experimental.pallas import tpu_sc as plsc

N_NODES = 10000
N_PAD = 10112
N_EDGES = 320000
D = 128

NUM_CORES = 2
NUM_SUBCORES = 16
NUM_TILES = NUM_CORES * NUM_SUBCORES
EDGES_PER_TILE = N_EDGES // NUM_TILES
BATCH = 96
CHUNKS = 105
EDGES_PAD_PER_TILE = CHUNKS * BATCH
PAD_DST = N_NODES + 64
ROWS_PER_SUBCORE = N_PAD // NUM_SUBCORES
LANES = 16

_MESH = plsc.VectorSubcoreMesh(core_axis_name="c", subcore_axis_name="s")

_SC_PARAMS = pltpu.CompilerParams()
if "needs_layout_passes" in pltpu.CompilerParams.__dataclass_fields__:
    _SC_PARAMS = dataclasses.replace(_SC_PARAMS, needs_layout_passes=False)


@functools.partial(
    pl.kernel,
    out_type=jax.ShapeDtypeStruct((NUM_TILES, 1, N_PAD), jnp.float32),
    mesh=_MESH,
    scratch_types=[
        pltpu.VMEM((EDGES_PER_TILE,), jnp.int32),
        pltpu.VMEM((N_PAD,), jnp.float32),
    ],
    compiler_params=_SC_PARAMS,
)
def _degree_kernel(dst_hbm, out_hbm, idx_v, deg_v):
    c = lax.axis_index("c")
    s = lax.axis_index("s")
    wid = c * NUM_SUBCORES + s

    @pl.loop(0, N_PAD // LANES)
    def _(i):
        deg_v[pl.ds(i * LANES, LANES)] = jnp.zeros((LANES,), jnp.float32)

    pltpu.sync_copy(dst_hbm.at[wid, 0], idx_v)
    ones = jnp.full((LANES,), 1.0, jnp.float32)

    @pl.loop(0, EDGES_PER_TILE // LANES)
    def _(i):
        idx = idx_v[pl.ds(i * LANES, LANES)]
        plsc.addupdate_scatter(deg_v, [idx], ones)

    pltpu.sync_copy(deg_v, out_hbm.at[wid, 0])


@functools.partial(
    pl.kernel,
    out_type=jax.ShapeDtypeStruct((NUM_CORES, N_PAD, D), jnp.float32),
    mesh=_MESH,
    scratch_types=[
        pltpu.VMEM((EDGES_PAD_PER_TILE,), jnp.int32),
        pltpu.VMEM((CHUNKS, BATCH), jnp.int32),
        pltpu.VMEM((BATCH, D), jnp.float32),
        pltpu.VMEM((BATCH, D), jnp.float32),
        pltpu.VMEM_SHARED((N_PAD, D), jnp.float32),
        pltpu.SemaphoreType.DMA,
        pltpu.SemaphoreType.DMA,
    ],
    compiler_params=_SC_PARAMS,
)
def _aggregate_kernel(g_hbm, src_hbm, dst_hbm, zero_hbm, out_hbm,
                      src_v, dst_v, rows0_v, rows1_v, acc_shared,
                      sg0, sg1):
    c = lax.axis_index("c")
    s = lax.axis_index("s")
    wid = c * NUM_SUBCORES + s
    row0 = s * ROWS_PER_SUBCORE

    pltpu.sync_copy(zero_hbm, acc_shared.at[pl.ds(row0, ROWS_PER_SUBCORE)])
    pltpu.sync_copy(src_hbm.at[wid, 0], src_v)
    pltpu.sync_copy(dst_hbm.at[wid], dst_v)
    plsc.subcore_barrier()

    def gather_sync(j, rows):
        pltpu.sync_copy(g_hbm.at[src_v.at[pl.ds(j * BATCH, BATCH)]], rows)

    def scatter_start(j, rows, sem):
        pltpu.async_copy(rows, acc_shared.at[dst_v.at[j]], sem, add=True)

    def scatter_wait(j, rows, sem):
        pltpu.make_async_copy(rows, acc_shared.at[dst_v.at[j]], sem).wait()

    @pl.loop(0, (CHUNKS - 1) // 2)
    def _(i):
        j = 2 * i

        @pl.when(i > 0)
        def _():
            scatter_wait(j, rows0_v, sg0)

        gather_sync(j, rows0_v)
        scatter_start(j, rows0_v, sg0)

        @pl.when(i > 0)
        def _():
            scatter_wait(j + 1, rows1_v, sg1)

        gather_sync(j + 1, rows1_v)
        scatter_start(j + 1, rows1_v, sg1)

    scatter_wait(CHUNKS - 1, rows0_v, sg0)
    gather_sync(CHUNKS - 1, rows0_v)
    scatter_start(CHUNKS - 1, rows0_v, sg0)
    scatter_wait(CHUNKS - 1, rows0_v, sg0)
    scatter_wait(CHUNKS - 1, rows1_v, sg1)

    plsc.subcore_barrier()
    pltpu.sync_copy(acc_shared.at[pl.ds(row0, ROWS_PER_SUBCORE)],
                    out_hbm.at[c, pl.ds(row0, ROWS_PER_SUBCORE)])


_PAD_BLOCK = 1264
_OUT_BLOCK = 2000


def _matmul_body(x_ref, w_ref, b_ref, h_ref):
    h_ref[...] = lax.dot_general(
        x_ref[...], w_ref[...], (((1,), (1,)), ((), ())),
        preferred_element_type=jnp.float32) + b_ref[...]


def _matmul(x, w, b2d):
    return pl.pallas_call(
        _matmul_body,
        grid=(N_PAD // _PAD_BLOCK,),
        in_specs=[
            pl.BlockSpec((_PAD_BLOCK, D), lambda i: (i, 0)),
            pl.BlockSpec((D, D), lambda i: (0, 0)),
            pl.BlockSpec((1, D), lambda i: (0, 0)),
        ],
        out_specs=pl.BlockSpec((_PAD_BLOCK, D), lambda i: (i, 0)),
        out_shape=jax.ShapeDtypeStruct((N_PAD, D), jnp.float32),
    )(x, w, b2d)


def _scale_body(pd_ref, h_ref, g_ref, r_ref):
    deg = jnp.sum(pd_ref[...], axis=1, keepdims=True) + 1.0
    r = lax.rsqrt(deg)
    r_ref[...] = r
    g_ref[...] = h_ref[...] * r


def _scale(pd_t, h):
    return pl.pallas_call(
        _scale_body,
        grid=(N_PAD // _PAD_BLOCK,),
        in_specs=[
            pl.BlockSpec((_PAD_BLOCK, NUM_TILES), lambda i: (i, 0)),
            pl.BlockSpec((_PAD_BLOCK, D), lambda i: (i, 0)),
        ],
        out_specs=[
            pl.BlockSpec((_PAD_BLOCK, D), lambda i: (i, 0)),
            pl.BlockSpec((_PAD_BLOCK, 1), lambda i: (i, 0)),
        ],
        out_shape=[
            jax.ShapeDtypeStruct((N_PAD, D), jnp.float32),
            jax.ShapeDtypeStruct((N_PAD, 1), jnp.float32),
        ],
    )(pd_t, h)


def _combine_body(p_ref, g_ref, r_ref, o_ref):
    o_ref[...] = (p_ref[0] + p_ref[1] + g_ref[...]) * r_ref[...]


def _combine(partials, g, r):
    return pl.pallas_call(
        _combine_body,
        grid=(N_NODES // _OUT_BLOCK,),
        in_specs=[
            pl.BlockSpec((NUM_CORES, _OUT_BLOCK, D), lambda i: (0, i, 0)),
            pl.BlockSpec((_OUT_BLOCK, D), lambda i: (i, 0)),
            pl.BlockSpec((_OUT_BLOCK, 1), lambda i: (i, 0)),
        ],
        out_specs=pl.BlockSpec((_OUT_BLOCK, D), lambda i: (i, 0)),
        out_shape=jax.ShapeDtypeStruct((N_NODES, D), jnp.float32),
    )(partials, g, r)


def kernel(x, edge_index, W, b):
    src = edge_index[0].astype(jnp.int32)
    dst = edge_index[1].astype(jnp.int32)
    dst_tiles = dst.reshape(NUM_TILES, 1, EDGES_PER_TILE)
    n_pad_edges = NUM_TILES * EDGES_PAD_PER_TILE - N_EDGES
    src_pad = jnp.concatenate(
        [src, jnp.zeros((n_pad_edges,), jnp.int32)])
    dst_pad = jnp.concatenate(
        [dst, jnp.full((n_pad_edges,), PAD_DST, jnp.int32)])
    src_chunks = src_pad.reshape(NUM_TILES, 1, EDGES_PAD_PER_TILE)
    dst_chunks = dst_pad.reshape(NUM_TILES, CHUNKS, BATCH)
    xp = jnp.pad(x, ((0, N_PAD - N_NODES), (0, 0)))

    partial_deg = _degree_kernel(dst_tiles)
    h = _matmul(xp, W, b.reshape(1, D))
    g, r = _scale(partial_deg.reshape(NUM_TILES, N_PAD).T, h)
    zeros = jnp.zeros((ROWS_PER_SUBCORE, D), jnp.float32)
    partials = _aggregate_kernel(g, src_chunks, dst_chunks, zeros)
    return _combine(partials, g, r)


# speedup vs baseline: 1.1539x; 1.1539x over previous
import dataclasses
import functools

import jax
import jax.numpy as jnp
from jax import lax
from jax.experimental import pallas as pl
from jax.experimental.pallas import tpu as pltpu
from jax.experimental.pallas import tpu_sc as plsc

N_NODES = 10000
N_PAD = 10112
N_EDGES = 320000
D = 128

NUM_CORES = 2
NUM_SUBCORES = 16
NUM_TILES = NUM_CORES * NUM_SUBCORES
EDGES_PER_TILE = N_EDGES // NUM_TILES
BATCH = 96
FAST_CORE = 0
CHUNKS_F = 143
CHUNKS_S = 67
EDGES_F = CHUNKS_F * BATCH
EDGES_S = CHUNKS_S * BATCH
PAD_DST = N_NODES + 64
ROWS_PER_SUBCORE = N_PAD // NUM_SUBCORES
LANES = 16

_MESH = plsc.VectorSubcoreMesh(core_axis_name="c", subcore_axis_name="s")

_SC_PARAMS = pltpu.CompilerParams()
if "needs_layout_passes" in pltpu.CompilerParams.__dataclass_fields__:
    _SC_PARAMS = dataclasses.replace(_SC_PARAMS, needs_layout_passes=False)


@functools.partial(
    pl.kernel,
    out_type=jax.ShapeDtypeStruct((NUM_TILES, 1, N_PAD), jnp.float32),
    mesh=_MESH,
    scratch_types=[
        pltpu.VMEM((EDGES_PER_TILE,), jnp.int32),
        pltpu.VMEM((N_PAD,), jnp.float32),
    ],
    compiler_params=_SC_PARAMS,
)
def _degree_kernel(dst_hbm, out_hbm, idx_v, deg_v):
    c = lax.axis_index("c")
    s = lax.axis_index("s")
    wid = c * NUM_SUBCORES + s

    @pl.loop(0, N_PAD // LANES)
    def _(i):
        deg_v[pl.ds(i * LANES, LANES)] = jnp.zeros((LANES,), jnp.float32)

    pltpu.sync_copy(dst_hbm.at[wid, 0], idx_v)
    ones = jnp.full((LANES,), 1.0, jnp.float32)

    @pl.loop(0, EDGES_PER_TILE // LANES)
    def _(i):
        idx = idx_v[pl.ds(i * LANES, LANES)]
        plsc.addupdate_scatter(deg_v, [idx], ones)

    pltpu.sync_copy(deg_v, out_hbm.at[wid, 0])


@functools.partial(
    pl.kernel,
    out_type=jax.ShapeDtypeStruct((NUM_CORES, N_PAD, D), jnp.float32),
    mesh=_MESH,
    scratch_types=[
        pltpu.VMEM((EDGES_F,), jnp.int32),
        pltpu.VMEM((1, BATCH), jnp.int32),
        pltpu.VMEM((1, BATCH), jnp.int32),
        pltpu.VMEM((BATCH, D), jnp.float32),
        pltpu.VMEM((BATCH, D), jnp.float32),
        pltpu.VMEM_SHARED((N_PAD, D), jnp.float32),
        pltpu.SemaphoreType.DMA,
        pltpu.SemaphoreType.DMA,
        pltpu.SemaphoreType.DMA,
        pltpu.SemaphoreType.DMA,
    ],
    compiler_params=_SC_PARAMS,
)
def _aggregate_kernel(g_hbm, srcf_hbm, dstf_hbm, srcs_hbm, dsts_hbm,
                      zero_hbm, out_hbm,
                      src_v, db0, db1, rows0_v, rows1_v, acc_shared,
                      sg0, sg1, sd0, sd1):
    c = lax.axis_index("c")
    s = lax.axis_index("s")
    row0 = s * ROWS_PER_SUBCORE

    pltpu.sync_copy(zero_hbm, acc_shared.at[pl.ds(row0, ROWS_PER_SUBCORE)])

    def run(src_hbm, dst_hbm, nchunks):
        pltpu.sync_copy(src_hbm.at[s, 0],
                        src_v.at[pl.ds(0, nchunks * BATCH)])
        plsc.subcore_barrier()

        def gather(j, rows, sem):
            pltpu.async_copy(g_hbm.at[src_v.at[pl.ds(j * BATCH, BATCH)]],
                             rows, sem)

        def gather_wait(j, rows, sem):
            pltpu.make_async_copy(
                g_hbm.at[src_v.at[pl.ds(j * BATCH, BATCH)]],
                rows, sem).wait()

        def didx(j, db, sem):
            pltpu.async_copy(dst_hbm.at[s, j], db, sem)

        def didx_wait(j, db, sem):
            pltpu.make_async_copy(dst_hbm.at[s, j], db, sem).wait()

        didx(0, db0, sd0)
        gather(0, rows0_v, sg0)

        @pl.loop(0, (nchunks - 1) // 2)
        def _(i):
            j = 2 * i
            didx(j + 1, db1, sd1)
            gather(j + 1, rows1_v, sg1)
            gather_wait(j, rows0_v, sg0)
            didx_wait(j, db0, sd0)
            pltpu.sync_copy(rows0_v, acc_shared.at[db0.at[0]], add=True)
            didx(j + 2, db0, sd0)
            gather(j + 2, rows0_v, sg0)
            gather_wait(j + 1, rows1_v, sg1)
            didx_wait(j + 1, db1, sd1)
            pltpu.sync_copy(rows1_v, acc_shared.at[db1.at[0]], add=True)

        gather_wait(nchunks - 1, rows0_v, sg0)
        didx_wait(nchunks - 1, db0, sd0)
        pltpu.sync_copy(rows0_v, acc_shared.at[db0.at[0]], add=True)

    @pl.when(c == FAST_CORE)
    def _():
        run(srcf_hbm, dstf_hbm, CHUNKS_F)

    @pl.when(c != FAST_CORE)
    def _():
        run(srcs_hbm, dsts_hbm, CHUNKS_S)

    plsc.subcore_barrier()
    pltpu.sync_copy(acc_shared.at[pl.ds(row0, ROWS_PER_SUBCORE)],
                    out_hbm.at[c, pl.ds(row0, ROWS_PER_SUBCORE)])


_PAD_BLOCK = 1264
_OUT_BLOCK = 2000


def _matmul_body(x_ref, w_ref, b_ref, h_ref):
    h_ref[...] = lax.dot_general(
        x_ref[...], w_ref[...], (((1,), (1,)), ((), ())),
        preferred_element_type=jnp.float32) + b_ref[...]


def _matmul(x, w, b2d):
    return pl.pallas_call(
        _matmul_body,
        grid=(N_PAD // _PAD_BLOCK,),
        in_specs=[
            pl.BlockSpec((_PAD_BLOCK, D), lambda i: (i, 0)),
            pl.BlockSpec((D, D), lambda i: (0, 0)),
            pl.BlockSpec((1, D), lambda i: (0, 0)),
        ],
        out_specs=pl.BlockSpec((_PAD_BLOCK, D), lambda i: (i, 0)),
        out_shape=jax.ShapeDtypeStruct((N_PAD, D), jnp.float32),
    )(x, w, b2d)


def _scale_body(pd_ref, h_ref, g_ref, r_ref):
    deg = jnp.sum(pd_ref[...], axis=1, keepdims=True) + 1.0
    r = lax.rsqrt(deg)
    r_ref[...] = r
    g_ref[...] = h_ref[...] * r


def _scale(pd_t, h):
    return pl.pallas_call(
        _scale_body,
        grid=(N_PAD // _PAD_BLOCK,),
        in_specs=[
            pl.BlockSpec((_PAD_BLOCK, NUM_TILES), lambda i: (i, 0)),
            pl.BlockSpec((_PAD_BLOCK, D), lambda i: (i, 0)),
        ],
        out_specs=[
            pl.BlockSpec((_PAD_BLOCK, D), lambda i: (i, 0)),
            pl.BlockSpec((_PAD_BLOCK, 1), lambda i: (i, 0)),
        ],
        out_shape=[
            jax.ShapeDtypeStruct((N_PAD, D), jnp.float32),
            jax.ShapeDtypeStruct((N_PAD, 1), jnp.float32),
        ],
    )(pd_t, h)


def _combine_body(p_ref, g_ref, r_ref, o_ref):
    o_ref[...] = (p_ref[0] + p_ref[1] + g_ref[...]) * r_ref[...]


def _combine(partials, g, r):
    return pl.pallas_call(
        _combine_body,
        grid=(N_NODES // _OUT_BLOCK,),
        in_specs=[
            pl.BlockSpec((NUM_CORES, _OUT_BLOCK, D), lambda i: (0, i, 0)),
            pl.BlockSpec((_OUT_BLOCK, D), lambda i: (i, 0)),
            pl.BlockSpec((_OUT_BLOCK, 1), lambda i: (i, 0)),
        ],
        out_specs=pl.BlockSpec((_OUT_BLOCK, D), lambda i: (i, 0)),
        out_shape=jax.ShapeDtypeStruct((N_NODES, D), jnp.float32),
    )(partials, g, r)


def kernel(x, edge_index, W, b):
    src = edge_index[0].astype(jnp.int32)
    dst = edge_index[1].astype(jnp.int32)
    dst_tiles = dst.reshape(NUM_TILES, 1, EDGES_PER_TILE)
    nf = NUM_SUBCORES * EDGES_F
    n_pad_edges = nf + NUM_SUBCORES * EDGES_S - N_EDGES
    src_pad = jnp.concatenate(
        [src, jnp.zeros((n_pad_edges,), jnp.int32)])
    dst_pad = jnp.concatenate(
        [dst, jnp.full((n_pad_edges,), PAD_DST, jnp.int32)])
    src_f = src_pad[:nf].reshape(NUM_SUBCORES, 1, EDGES_F)
    dst_f = dst_pad[:nf].reshape(NUM_SUBCORES, CHUNKS_F, 1, BATCH)
    src_s = src_pad[nf:].reshape(NUM_SUBCORES, 1, EDGES_S)
    dst_s = dst_pad[nf:].reshape(NUM_SUBCORES, CHUNKS_S, 1, BATCH)
    xp = jnp.pad(x, ((0, N_PAD - N_NODES), (0, 0)))

    partial_deg = _degree_kernel(dst_tiles)
    h = _matmul(xp, W, b.reshape(1, D))
    g, r = _scale(partial_deg.reshape(NUM_TILES, N_PAD).T, h)
    zeros = jnp.zeros((ROWS_PER_SUBCORE, D), jnp.float32)
    partials = _aggregate_kernel(g, src_f, dst_f, src_s, dst_s, zeros)
    return _combine(partials, g, r)


# speedup vs baseline: 1.4661x; 1.2706x over previous
import dataclasses
import functools

import jax
import jax.numpy as jnp
from jax import lax
from jax.experimental import pallas as pl
from jax.experimental.pallas import tpu as pltpu
from jax.experimental.pallas import tpu_sc as plsc

N_NODES = 10000
N_PAD = 10112
N_DEG = 10240
N_EDGES = 320000
D = 128

NUM_CORES = 2
NUM_SUBCORES = 16
EDGES_PER_SUBCORE = N_EDGES // NUM_SUBCORES
BATCH = 80
FAST_CORE = 0
CHUNKS_F = 159
CHUNKS_S = 91
EDGES_F = CHUNKS_F * BATCH
EDGES_S = CHUNKS_S * BATCH
ROWS_PER_SUBCORE = N_PAD // NUM_SUBCORES
LANES = 16

_MESH = plsc.VectorSubcoreMesh(core_axis_name="c", subcore_axis_name="s")

_SC_PARAMS = pltpu.CompilerParams()
if "needs_layout_passes" in pltpu.CompilerParams.__dataclass_fields__:
    _SC_PARAMS = dataclasses.replace(_SC_PARAMS, needs_layout_passes=False)


@functools.partial(
    pl.kernel,
    out_type=jax.ShapeDtypeStruct((NUM_CORES, N_DEG // D, D), jnp.float32),
    mesh=_MESH,
    scratch_types=[
        pltpu.VMEM((EDGES_PER_SUBCORE,), jnp.int32),
        pltpu.VMEM((N_DEG // D, D), jnp.float32),
        pltpu.VMEM((1, N_DEG // D), jnp.int32),
        pltpu.VMEM_SHARED((N_DEG // D, D), jnp.float32),
    ],
    compiler_params=_SC_PARAMS,
)
def _degree_kernel(dst_hbm, ident_hbm, out_hbm, idx_v, deg_v, ident_v,
                   deg_shared):
    c = lax.axis_index("c")
    s = lax.axis_index("s")

    @pl.loop(0, N_DEG // D)
    def _(rr):
        @pl.loop(0, D // LANES)
        def _(cc):
            deg_v[rr, pl.ds(cc * LANES, LANES)] = jnp.zeros(
                (LANES,), jnp.float32)

    @pl.when(s < 5)
    def _():
        pltpu.sync_copy(deg_v.at[pl.ds(s * 16, 16)],
                        deg_shared.at[pl.ds(s * 16, 16)])

    pltpu.sync_copy(ident_hbm, ident_v)
    pltpu.sync_copy(dst_hbm.at[s, 0], idx_v)
    ones = jnp.full((LANES,), 1.0, jnp.float32)

    @pl.loop(0, EDGES_PER_SUBCORE // LANES)
    def _(i):
        idx = idx_v[pl.ds(i * LANES, LANES)]
        plsc.addupdate_scatter(deg_v, [idx >> 7, idx & 127], ones)

    plsc.subcore_barrier()

    pltpu.sync_copy(deg_v, deg_shared.at[ident_v.at[0]], add=True)

    plsc.subcore_barrier()

    @pl.when(s < 5)
    def _():
        pltpu.sync_copy(deg_shared.at[pl.ds(s * 16, 16)],
                        out_hbm.at[c, pl.ds(s * 16, 16)])


@functools.partial(
    pl.kernel,
    out_type=jax.ShapeDtypeStruct((NUM_CORES, N_PAD, D), jnp.float32),
    mesh=_MESH,
    scratch_types=[
        pltpu.VMEM((EDGES_F,), jnp.int32),
        pltpu.VMEM((1, BATCH), jnp.int32),
        pltpu.VMEM((1, BATCH), jnp.int32),
        pltpu.VMEM((BATCH, D), jnp.float32),
        pltpu.VMEM((BATCH, D), jnp.float32),
        pltpu.VMEM_SHARED((N_PAD, D), jnp.float32),
        pltpu.SemaphoreType.DMA,
        pltpu.SemaphoreType.DMA,
        pltpu.SemaphoreType.DMA,
        pltpu.SemaphoreType.DMA,
    ],
    compiler_params=_SC_PARAMS,
)
def _aggregate_kernel(g_hbm, srcf_hbm, dstf_hbm, srcs_hbm, dsts_hbm,
                      zero_hbm, out_hbm,
                      src_v, db0, db1, rows0_v, rows1_v, acc_shared,
                      sg0, sg1, sd0, sd1):
    c = lax.axis_index("c")
    s = lax.axis_index("s")
    row0 = s * ROWS_PER_SUBCORE

    @pl.when(c == FAST_CORE)
    def _():
        pltpu.sync_copy(g_hbm.at[pl.ds(row0, ROWS_PER_SUBCORE)],
                        acc_shared.at[pl.ds(row0, ROWS_PER_SUBCORE)])

    @pl.when(c != FAST_CORE)
    def _():
        pltpu.sync_copy(zero_hbm,
                        acc_shared.at[pl.ds(row0, ROWS_PER_SUBCORE)])

    def run(src_hbm, dst_hbm, nchunks):
        pltpu.sync_copy(src_hbm.at[s, 0],
                        src_v.at[pl.ds(0, nchunks * BATCH)])
        plsc.subcore_barrier()

        def gather(j, rows, sem):
            pltpu.async_copy(g_hbm.at[src_v.at[pl.ds(j * BATCH, BATCH)]],
                             rows, sem)

        def gather_wait(j, rows, sem):
            pltpu.make_async_copy(
                g_hbm.at[src_v.at[pl.ds(j * BATCH, BATCH)]],
                rows, sem).wait()

        def didx(j, db, sem):
            pltpu.async_copy(dst_hbm.at[s, j], db, sem)

        def didx_wait(j, db, sem):
            pltpu.make_async_copy(dst_hbm.at[s, j], db, sem).wait()

        didx(0, db0, sd0)
        gather(0, rows0_v, sg0)

        @pl.loop(0, (nchunks - 1) // 2)
        def _(i):
            j = 2 * i
            didx(j + 1, db1, sd1)
            gather(j + 1, rows1_v, sg1)
            gather_wait(j, rows0_v, sg0)
            didx_wait(j, db0, sd0)
            pltpu.sync_copy(rows0_v, acc_shared.at[db0.at[0]], add=True)
            didx(j + 2, db0, sd0)
            gather(j + 2, rows0_v, sg0)
            gather_wait(j + 1, rows1_v, sg1)
            didx_wait(j + 1, db1, sd1)
            pltpu.sync_copy(rows1_v, acc_shared.at[db1.at[0]], add=True)

        gather_wait(nchunks - 1, rows0_v, sg0)
        didx_wait(nchunks - 1, db0, sd0)
        pltpu.sync_copy(rows0_v, acc_shared.at[db0.at[0]], add=True)

    @pl.when(c == FAST_CORE)
    def _():
        run(srcf_hbm, dstf_hbm, CHUNKS_F)

    @pl.when(c != FAST_CORE)
    def _():
        run(srcs_hbm, dsts_hbm, CHUNKS_S)

    plsc.subcore_barrier()
    pltpu.sync_copy(acc_shared.at[pl.ds(row0, ROWS_PER_SUBCORE)],
                    out_hbm.at[c, pl.ds(row0, ROWS_PER_SUBCORE)])


_BLOCK = 2000


def _matmul_body(x_ref, w_ref, b_ref, h_ref):
    h_ref[...] = lax.dot_general(
        x_ref[...], w_ref[...], (((1,), (1,)), ((), ())),
        preferred_element_type=jnp.float32) + b_ref[...]


def _matmul(x, w, b2d):
    return pl.pallas_call(
        _matmul_body,
        grid=(N_NODES // _BLOCK,),
        in_specs=[
            pl.BlockSpec((_BLOCK, D), lambda i: (i, 0)),
            pl.BlockSpec((D, D), lambda i: (0, 0)),
            pl.BlockSpec((1, D), lambda i: (0, 0)),
        ],
        out_specs=pl.BlockSpec((_BLOCK, D), lambda i: (i, 0)),
        out_shape=jax.ShapeDtypeStruct((N_NODES, D), jnp.float32),
    )(x, w, b2d)


def _scale_body(deg_ref, h_ref, g_ref, r_ref):
    r = lax.rsqrt(deg_ref[...] + 1.0)
    r_ref[...] = r
    g_ref[...] = h_ref[...] * r


def _scale(deg_col, h):
    return pl.pallas_call(
        _scale_body,
        grid=(N_NODES // _BLOCK,),
        in_specs=[
            pl.BlockSpec((_BLOCK, 1), lambda i: (i, 0)),
            pl.BlockSpec((_BLOCK, D), lambda i: (i, 0)),
        ],
        out_specs=[
            pl.BlockSpec((_BLOCK, D), lambda i: (i, 0)),
            pl.BlockSpec((_BLOCK, 1), lambda i: (i, 0)),
        ],
        out_shape=[
            jax.ShapeDtypeStruct((N_PAD, D), jnp.float32),
            jax.ShapeDtypeStruct((N_NODES, 1), jnp.float32),
        ],
    )(deg_col, h)


def _combine_body(p_ref, r_ref, o_ref):
    o_ref[...] = (p_ref[0] + p_ref[1]) * r_ref[...]


def _combine(partials, r):
    return pl.pallas_call(
        _combine_body,
        grid=(N_NODES // _BLOCK,),
        in_specs=[
            pl.BlockSpec((NUM_CORES, _BLOCK, D), lambda i: (0, i, 0)),
            pl.BlockSpec((_BLOCK, 1), lambda i: (i, 0)),
        ],
        out_specs=pl.BlockSpec((_BLOCK, D), lambda i: (i, 0)),
        out_shape=jax.ShapeDtypeStruct((N_NODES, D), jnp.float32),
    )(partials, r)


def kernel(x, edge_index, W, b):
    edges = edge_index.astype(jnp.int32)
    src, dst = edges[0], edges[1]
    dst_tiles = dst.reshape(NUM_SUBCORES, 1, EDGES_PER_SUBCORE)
    nf = NUM_SUBCORES * EDGES_F
    src_f = src[:nf].reshape(NUM_SUBCORES, 1, EDGES_F)
    dst_f = dst[:nf].reshape(NUM_SUBCORES, CHUNKS_F, 1, BATCH)
    src_s = src[nf:].reshape(NUM_SUBCORES, 1, EDGES_S)
    dst_s = dst[nf:].reshape(NUM_SUBCORES, CHUNKS_S, 1, BATCH)
    ident = jnp.arange(N_DEG // D, dtype=jnp.int32).reshape(1, N_DEG // D)

    deg2 = _degree_kernel(dst_tiles, ident)
    h = _matmul(x, W, b.reshape(1, D))
    deg_col = deg2[0].reshape(N_DEG)[:N_NODES, None]
    g, r = _scale(deg_col, h)
    zeros = jnp.zeros((ROWS_PER_SUBCORE, D), jnp.float32)
    partials = _aggregate_kernel(g, src_f, dst_f, src_s, dst_s, zeros)
    return _combine(partials, r)
